# baseline (speedup 1.0000x reference)
import functools

import jax
import jax.numpy as jnp
from jax.experimental import pallas as pl
from jax.experimental.pallas import tpu as pltpu

ALPHA = 0.2
LOG2E = 1.4426950408889634


def _proj_kernel(nhid, x_ref, w_ref, a_ref, whext_ref, u_ref,
                 v_ref, ep_ref, en_ref, meanwh_ref):
    wh = jnp.dot(x_ref[...], w_ref[...], preferred_element_type=jnp.float32)
    a_all = a_ref[...]
    esrc = jnp.dot(wh, a_all[:nhid, :], preferred_element_type=jnp.float32)
    edst = jnp.dot(wh, a_all[nhid:, :], preferred_element_type=jnp.float32)
    t = esrc + jnp.max(edst)
    m = jnp.where(t >= 0, t, ALPHA * t)
    u_ref[...] = (esrc - m) * LOG2E
    v_ref[...] = (ALPHA * esrc - m) * LOG2E
    ep_ref[...] = edst * LOG2E
    en_ref[...] = (ALPHA * LOG2E) * edst
    whext_ref[:, :nhid] = wh
    whext_ref[:, nhid:] = jnp.ones_like(whext_ref[:, nhid:])
    meanwh_ref[...] = jnp.mean(wh, axis=0, keepdims=True)


def _flash_kernel(nhid, br, half, u_ref, v_ref, ept_ref, ent_ref,
                  adja_ref, adjb_ref, whext_ref, meanwh_ref,
                  outa_ref, outb_ref):
    i = pl.program_id(0)
    ept = ept_ref[...]
    ent = ent_ref[...]
    whext = whext_ref[...]
    meanwh = meanwh_ref[...]

    def one_half(adj_ref, out_ref, row0):
        u = u_ref[pl.ds(row0, br), :]
        v = v_ref[pl.ds(row0, br), :]
        val = jnp.maximum(u + ept, v + ent)
        p = adj_ref[...] * jnp.exp2(val)
        h_ext = jnp.dot(p, whext, preferred_element_type=jnp.float32)
        l = h_ext[:, nhid:nhid + 1]
        h = h_ext[:, :nhid]
        h = jnp.where(l > 0, h / l, meanwh)
        out_ref[...] = jnp.where(h > 0, h, jnp.exp(h) - 1.0)

    one_half(adja_ref, outa_ref, i * br)
    one_half(adjb_ref, outb_ref, i * br + half)


def kernel(x, adj, W, a):
    n, _ = x.shape
    nhid = W.shape[1]
    f32 = jnp.float32

    whext, u, v, ep, en, meanwh = pl.pallas_call(
        functools.partial(_proj_kernel, nhid),
        out_shape=[
            jax.ShapeDtypeStruct((n, nhid + 1), f32),
            jax.ShapeDtypeStruct((n, 1), f32),
            jax.ShapeDtypeStruct((n, 1), f32),
            jax.ShapeDtypeStruct((n, 1), f32),
            jax.ShapeDtypeStruct((n, 1), f32),
            jax.ShapeDtypeStruct((1, nhid), f32),
        ],
    )(x, W, a)

    ept = ep.reshape(1, n)
    ent = en.reshape(1, n)

    half = n // 2
    br = 200 if half % 200 == 0 else half
    num_steps = half // br
    nblocks_half = num_steps

    out_a, out_b = pl.pallas_call(
        functools.partial(_flash_kernel, nhid, br, half),
        grid=(num_steps,),
        in_specs=[
            pl.BlockSpec((n, 1), lambda i: (0, 0)),
            pl.BlockSpec((n, 1), lambda i: (0, 0)),
            pl.BlockSpec((1, n), lambda i: (0, 0)),
            pl.BlockSpec((1, n), lambda i: (0, 0)),
            pl.BlockSpec((br, n), lambda i: (i, 0)),
            pl.BlockSpec((br, n),
                         lambda i: (i + nblocks_half, 0)),
            pl.BlockSpec((n, nhid + 1), lambda i: (0, 0)),
            pl.BlockSpec((1, nhid), lambda i: (0, 0)),
        ],
        out_specs=[
            pl.BlockSpec((br, nhid), lambda i: (i, 0)),
            pl.BlockSpec((br, nhid), lambda i: (i, 0)),
        ],
        out_shape=[
            jax.ShapeDtypeStruct((half, nhid), f32),
            jax.ShapeDtypeStruct((half, nhid), f32),
        ],
        compiler_params=pltpu.CompilerParams(
            dimension_semantics=("arbitrary",),
        ),
    )(u, v, ept, ent, adj, adj, whext, meanwh)
    return jnp.concatenate([out_a, out_b], axis=0)


# speedup vs baseline: 1.0217x; 1.0217x over previous
import functools

import jax
import jax.numpy as jnp
from jax.experimental import pallas as pl
from jax.experimental.pallas import tpu as pltpu

ALPHA = 0.2
LOG2E = 1.4426950408889634


def _proj_kernel(nhid, x_ref, w_ref, a_ref, whext_ref, u_ref,
                 v_ref, ep_ref, en_ref, meanwh_ref):
    wh = jnp.dot(x_ref[...], w_ref[...], preferred_element_type=jnp.float32)
    a_all = a_ref[...]
    esrc = jnp.dot(wh, a_all[:nhid, :], preferred_element_type=jnp.float32)
    edst = jnp.dot(wh, a_all[nhid:, :], preferred_element_type=jnp.float32)
    t = esrc + jnp.max(edst)
    m = jnp.where(t >= 0, t, ALPHA * t)
    u_ref[...] = (esrc - m) * LOG2E
    v_ref[...] = (ALPHA * esrc - m) * LOG2E
    ep_ref[...] = edst * LOG2E
    en_ref[...] = (ALPHA * LOG2E) * edst
    whext_ref[:, :nhid] = wh
    whext_ref[:, nhid:] = jnp.ones_like(whext_ref[:, nhid:])
    meanwh_ref[...] = jnp.mean(wh, axis=0, keepdims=True)


def _flash_kernel(nhid, u_ref, v_ref, ept_ref, ent_ref, adj_ref,
                  whext_ref, meanwh_ref, out_ref):
    val = jnp.maximum(u_ref[...] + ept_ref[...], v_ref[...] + ent_ref[...])
    p = adj_ref[...] * jnp.exp2(val)
    h_ext = jnp.dot(p, whext_ref[...], preferred_element_type=jnp.float32)
    l = h_ext[:, nhid:nhid + 1]
    h = h_ext[:, :nhid]
    h = jnp.where(l > 0, h / l, meanwh_ref[...])
    out_ref[...] = jnp.where(h > 0, h, jnp.exp(h) - 1.0)


def kernel(x, adj, W, a):
    n, _ = x.shape
    nhid = W.shape[1]
    f32 = jnp.float32

    whext, u, v, ep, en, meanwh = pl.pallas_call(
        functools.partial(_proj_kernel, nhid),
        out_shape=[
            jax.ShapeDtypeStruct((n, nhid + 1), f32),
            jax.ShapeDtypeStruct((n, 1), f32),
            jax.ShapeDtypeStruct((n, 1), f32),
            jax.ShapeDtypeStruct((n, 1), f32),
            jax.ShapeDtypeStruct((n, 1), f32),
            jax.ShapeDtypeStruct((1, nhid), f32),
        ],
    )(x, W, a)

    ept = ep.reshape(1, n)
    ent = en.reshape(1, n)

    br = 400 if n % 400 == 0 else n
    num_rb = n // br

    out = pl.pallas_call(
        functools.partial(_flash_kernel, nhid),
        grid=(num_rb,),
        in_specs=[
            pl.BlockSpec((br, 1), lambda i: (i, 0)),
            pl.BlockSpec((br, 1), lambda i: (i, 0)),
            pl.BlockSpec((1, n), lambda i: (0, 0)),
            pl.BlockSpec((1, n), lambda i: (0, 0)),
            pl.BlockSpec((br, n), lambda i: (i, 0)),
            pl.BlockSpec((n, nhid + 1), lambda i: (0, 0)),
            pl.BlockSpec((1, nhid), lambda i: (0, 0)),
        ],
        out_specs=pl.BlockSpec((br, nhid), lambda i: (i, 0)),
        out_shape=jax.ShapeDtypeStruct((n, nhid), f32),
        compiler_params=pltpu.CompilerParams(
            dimension_semantics=("parallel",),
        ),
    )(u, v, ept, ent, adj, whext, meanwh)
    return out


# speedup vs baseline: 1.0319x; 1.0100x over previous
import functools

import jax
import jax.numpy as jnp
from jax.experimental import pallas as pl
from jax.experimental.pallas import tpu as pltpu

ALPHA = 0.2
LOG2E = 1.4426950408889634


def _proj_kernel(nhid, x_ref, w_ref, a_ref, whext_ref, u_ref,
                 v_ref, ep_ref, en_ref, meanwh_ref):
    wh = jnp.dot(x_ref[...], w_ref[...], preferred_element_type=jnp.float32)
    a_all = a_ref[...]
    esrc = jnp.dot(wh, a_all[:nhid, :], preferred_element_type=jnp.float32)
    edst = jnp.dot(wh, a_all[nhid:, :], preferred_element_type=jnp.float32)
    t = esrc + jnp.max(edst)
    m = jnp.where(t >= 0, t, ALPHA * t)
    u_ref[...] = (esrc - m) * LOG2E
    v_ref[...] = (ALPHA * esrc - m) * LOG2E
    ep_ref[...] = edst * LOG2E
    en_ref[...] = (ALPHA * LOG2E) * edst
    whext_ref[:, :nhid] = wh.astype(jnp.bfloat16)
    whext_ref[:, nhid:] = jnp.ones_like(whext_ref[:, nhid:])
    meanwh_ref[...] = jnp.mean(wh, axis=0, keepdims=True)


def _flash_kernel(nhid, u_ref, v_ref, ept_ref, ent_ref, adj_ref,
                  whext_ref, meanwh_ref, out_ref):
    val = jnp.maximum(u_ref[...] + ept_ref[...], v_ref[...] + ent_ref[...])
    p = (adj_ref[...] * jnp.exp2(val)).astype(jnp.bfloat16)
    h_ext = jnp.dot(p, whext_ref[...], preferred_element_type=jnp.float32)
    l = h_ext[:, nhid:nhid + 1]
    h = h_ext[:, :nhid]
    h = jnp.where(l > 0, h / l, meanwh_ref[...])
    out_ref[...] = jnp.where(h > 0, h, jnp.exp(h) - 1.0)


def kernel(x, adj, W, a):
    n, _ = x.shape
    nhid = W.shape[1]
    f32 = jnp.float32

    whext, u, v, ep, en, meanwh = pl.pallas_call(
        functools.partial(_proj_kernel, nhid),
        out_shape=[
            jax.ShapeDtypeStruct((n, nhid + 1), jnp.bfloat16),
            jax.ShapeDtypeStruct((n, 1), f32),
            jax.ShapeDtypeStruct((n, 1), f32),
            jax.ShapeDtypeStruct((n, 1), f32),
            jax.ShapeDtypeStruct((n, 1), f32),
            jax.ShapeDtypeStruct((1, nhid), f32),
        ],
    )(x, W, a)

    ept = ep.reshape(1, n)
    ent = en.reshape(1, n)

    br = 400 if n % 400 == 0 else n
    num_rb = n // br

    out = pl.pallas_call(
        functools.partial(_flash_kernel, nhid),
        grid=(num_rb,),
        in_specs=[
            pl.BlockSpec((br, 1), lambda i: (i, 0)),
            pl.BlockSpec((br, 1), lambda i: (i, 0)),
            pl.BlockSpec((1, n), lambda i: (0, 0)),
            pl.BlockSpec((1, n), lambda i: (0, 0)),
            pl.BlockSpec((br, n), lambda i: (i, 0)),
            pl.BlockSpec((n, nhid + 1), lambda i: (0, 0)),
            pl.BlockSpec((1, nhid), lambda i: (0, 0)),
        ],
        out_specs=pl.BlockSpec((br, nhid), lambda i: (i, 0)),
        out_shape=jax.ShapeDtypeStruct((n, nhid), f32),
        compiler_params=pltpu.CompilerParams(
            dimension_semantics=("parallel",),
        ),
    )(u, v, ept, ent, adj, whext, meanwh)
    return out
